# baseline (speedup 1.0000x reference)
import functools

import jax
import jax.numpy as jnp
from jax import lax
from jax.experimental import pallas as pl
from jax.experimental.pallas import tpu as pltpu
from jax.experimental.pallas import tpu_sc as plsc

D = 64
NC = 2
NS = 16
NW = NC * NS
K = 2
CHUNK = K * 128
NBUF = 4


@functools.lru_cache(maxsize=None)
def _make_gather(n_rounds: int):
    mesh = plsc.VectorSubcoreMesh(core_axis_name="c", subcore_axis_name="s")
    n_chunks = n_rounds * NBUF
    b_total = NW * n_chunks * CHUNK
    assert n_rounds % 2 == 0 and n_rounds >= 4

    @functools.partial(
        pl.kernel,
        mesh=mesh,
        out_type=jax.ShapeDtypeStruct((b_total, 128), jnp.float32),
        scratch_types=[
            pltpu.VMEM((2, NBUF, K, 128), jnp.int32),
            pltpu.VMEM((NBUF, CHUNK, D), jnp.float32),
            [pltpu.SemaphoreType.DMA] * NBUF,
            [pltpu.SemaphoreType.DMA] * NBUF,
            [pltpu.SemaphoreType.DMA] * 2,
        ],
        compiler_params=pltpu.CompilerParams(use_tc_tiling_on_sc=False),
    )
    def gather(idx_hbm, table_hbm, out_hbm, idx_v, rows_v, gsems, ssems,
               isems):
        wid = lax.axis_index("s") * NC + lax.axis_index("c")

        def fire_idx(r, p):
            pltpu.async_copy(idx_hbm.at[wid, r], idx_v.at[p], isems[p])

        def wait_idx(r, p):
            pltpu.make_async_copy(idx_hbm.at[wid, r], idx_v.at[p],
                                  isems[p]).wait()

        def fire_chunk(b, p):
            for j in range(K):
                pltpu.async_copy(
                    table_hbm.at[idx_v.at[p, b, j]],
                    rows_v.at[b, pl.ds(j * 128, 128)],
                    gsems[b],
                )

        def drain_and_store(c, b, p):
            for j in range(K):
                pltpu.make_async_copy(
                    table_hbm.at[idx_v.at[p, b, j]],
                    rows_v.at[b, pl.ds(j * 128, 128)],
                    gsems[b],
                ).wait()
            base = (wid * n_chunks + c) * CHUNK
            pltpu.async_copy(rows_v.at[b],
                             out_hbm.at[pl.ds(base, CHUNK), pl.ds(0, D)],
                             ssems[b])

        def wait_store(c, b):
            base = (wid * n_chunks + c) * CHUNK
            pltpu.make_async_copy(rows_v.at[b],
                                  out_hbm.at[pl.ds(base, CHUNK), pl.ds(0, D)],
                                  ssems[b]).wait()

        def round_body(r, p, first):
            wait_idx(r, p)
            rn = jnp.minimum(r + 1, n_rounds - 1)
            fire_idx(rn, 1 - p)
            c0 = r * NBUF
            for b in range(NBUF):
                if not first:
                    wait_store(c0 - NBUF + b, b)
                fire_chunk(b, p)
            for b in range(NBUF):
                drain_and_store(c0 + b, b, p)

        fire_idx(0, 0)
        round_body(0, 0, True)
        round_body(1, 1, False)

        def body(i, carry):
            round_body(2 * i, 0, False)
            round_body(2 * i + 1, 1, False)
            return carry

        lax.fori_loop(1, n_rounds // 2, body, 0)

        wait_idx(n_rounds - 1, 0)
        for b in range(NBUF):
            wait_store(n_chunks - NBUF + b, b)

    return gather


def kernel(x, table):
    orig_shape = x.shape
    xf = x.reshape(-1).astype(jnp.int32)
    b = xf.shape[0]
    per_call = NW * CHUNK * NBUF * 2
    n_rounds = 2 * (-(-b // per_call))
    pad = n_rounds * NBUF * CHUNK * NW - b
    if pad:
        xf = jnp.concatenate([xf, jnp.zeros((pad,), jnp.int32)])
    idx = xf.reshape(NW, n_rounds, NBUF, K, 128)
    out = _make_gather(n_rounds)(idx, table)
    out = out[:, :D]
    if pad:
        out = out[:b]
    return out.reshape(*orig_shape, D)
